# baseline (speedup 1.0000x reference)
import functools

import jax
import jax.numpy as jnp
from jax import lax
from jax.experimental import pallas as pl
from jax.experimental.pallas import tpu as pltpu
from jax.experimental.pallas import tpu_sc as plsc

_N = 200
_L = 16
_NC = 2
_NS = 16
_NW = _NC * _NS
_ROWS = 7
_CHUNKS = (_N + _L - 1) // _L
_TOL = 0.01
_NACC = 4

_A0, _A1, _A2 = 0.69306216, -0.16045562, 0.00981685
_B1, _B2, _B3, _B4 = 0.4876811, 0.19451932, 0.03766924, 0.01167206


def _loss_body(sl_hbm, lab_hbm, out_hbm, sl_v, lab_v, acc_v, buf_v,
               part_sh, out_v, sem):
    cid = lax.axis_index("c")
    sid = lax.axis_index("s")
    w = cid * _NS + sid

    cp1 = pltpu.async_copy(sl_hbm, sl_v, sem)
    cp2 = pltpu.async_copy(lab_hbm, lab_v, sem)
    cp1.wait()
    cp2.wait()

    jbase = lax.iota(jnp.int32, _L)
    sidvec = jnp.broadcast_to(sid, (_L,)).astype(jnp.int32)

    def row_body(r, accs):
        i = w + _NW * r
        rrow = cid + 2 * r
        sl_i = sl_v[rrow].at[sidvec].get(mode="promise_in_bounds")
        lab_i = lab_v[rrow].at[sidvec].get(mode="promise_in_bounds")
        ivalid = jnp.maximum(jnp.sign(
            jnp.full((_L,), 199.5, jnp.float32) -
            jnp.full((_L,), i, jnp.int32).astype(jnp.float32)), 0.0)

        accs = list(accs)
        for c in range(_CHUNKS):
            sl_j = sl_v[c]
            lab_j = lab_v[c]
            d = sl_i - sl_j
            mval = jnp.maximum(jnp.sign(lab_i - lab_j - _TOL), 0.0)
            t = jnp.abs(d)
            num = _A0 + t * (_A1 + t * _A2)
            den = 1.0 + t * (_B1 + t * (_B2 + t * (_B3 + t * _B4)))
            val = jnp.minimum(d, 0.0) - num / den
            accs[c % _NACC] = accs[c % _NACC] + (mval * ivalid) * val
        return tuple(accs)

    zero = jnp.zeros((_L,), jnp.float32)
    accs = lax.fori_loop(0, _ROWS, row_body, (zero,) * _NACC)
    acc = (accs[0] + accs[1]) + (accs[2] + accs[3])

    for step in (1, 2, 4, 8):
        acc = acc + acc.at[jbase ^ step].get(mode="promise_in_bounds")

    acc_v[...] = acc
    pltpu.sync_copy(acc_v, part_sh.at[sid])
    plsc.subcore_barrier()

    @pl.when(sid == 0)
    def _():
        pltpu.sync_copy(part_sh, buf_v)
        tot = jnp.zeros((_L,), jnp.float32)
        for k in range(_NS):
            tot = tot + buf_v[k]
        out_v[...] = tot
        pltpu.sync_copy(out_v, out_hbm.at[cid])


@jax.jit
def _ranking_loss(sl_pad, lab_pad):
    mesh = plsc.VectorSubcoreMesh(core_axis_name="c", subcore_axis_name="s")
    run = functools.partial(
        pl.kernel, mesh=mesh,
        out_type=jax.ShapeDtypeStruct((_NC, _L), jnp.float32),
        scratch_types=[
            pltpu.VMEM((16, _L), jnp.float32),
            pltpu.VMEM((16, _L), jnp.float32),
            pltpu.VMEM((_L,), jnp.float32),
            pltpu.VMEM((_NS, _L), jnp.float32),
            pltpu.VMEM_SHARED((_NS, _L), jnp.float32),
            pltpu.VMEM((_L,), jnp.float32),
            pltpu.SemaphoreType.DMA,
        ],
    )(_loss_body)
    return run(sl_pad, lab_pad)


def kernel(logits, labels):
    pad = 16 * _L - _N
    sl_pad = jnp.pad(logits, (0, pad)).reshape(16, _L)
    lab_pad = jnp.pad(labels, (0, pad), constant_values=2.0).reshape(16, _L)
    out = _ranking_loss(sl_pad, lab_pad)
    return out[0, 0] + out[1, 0]


# speedup vs baseline: 1.0009x; 1.0009x over previous
import functools

import jax
import jax.numpy as jnp
from jax import lax
from jax.experimental import pallas as pl
from jax.experimental.pallas import tpu as pltpu
from jax.experimental.pallas import tpu_sc as plsc

_N = 200
_L = 16
_NC = 2
_NS = 16
_NW = _NC * _NS
_ROWS = 7
_CHUNKS = (_N + _L - 1) // _L
_TOL = 0.01
_NACC = 4

_A0, _A1, _A2 = 0.69306216, -0.16045562, 0.00981685
_B1, _B2, _B3, _B4 = 0.4876811, 0.19451932, 0.03766924, 0.01167206


def _loss_body(packed_hbm, out_hbm, pk_v, acc_v, buf_v, part_sh, out_v):
    cid = lax.axis_index("c")
    sid = lax.axis_index("s")

    pltpu.sync_copy(packed_hbm, pk_v)

    jbase = lax.iota(jnp.int32, _L)
    sidvec = jnp.broadcast_to(sid, (_L,)).astype(jnp.int32)

    def row_body(r, accs):
        rrow = cid + 2 * r
        sl_i = pk_v[rrow].at[sidvec].get(mode="promise_in_bounds")
        lab_i = pk_v[32 + rrow].at[sidvec].get(mode="promise_in_bounds")

        accs = list(accs)
        for c in range(_CHUNKS):
            sl_j = pk_v[c]
            lab_j = pk_v[16 + c]
            d = sl_i - sl_j
            mval = jnp.maximum(jnp.sign(lab_i - lab_j - _TOL), 0.0)
            t = jnp.abs(d)
            num = _A0 + t * (_A1 + t * _A2)
            den = 1.0 + t * (_B1 + t * (_B2 + t * (_B3 + t * _B4)))
            val = jnp.minimum(d, 0.0) - num / den
            accs[c % _NACC] = accs[c % _NACC] + mval * val
        return tuple(accs)

    zero = jnp.zeros((_L,), jnp.float32)
    accs = lax.fori_loop(0, _ROWS, row_body, (zero,) * _NACC)
    acc = (accs[0] + accs[1]) + (accs[2] + accs[3])

    for step in (1, 2, 4, 8):
        acc = acc + acc.at[jbase ^ step].get(mode="promise_in_bounds")

    acc_v[...] = acc
    pltpu.sync_copy(acc_v, part_sh.at[sid])
    plsc.subcore_barrier()

    @pl.when(sid == 0)
    def _():
        pltpu.sync_copy(part_sh, buf_v)
        tot = jnp.zeros((_L,), jnp.float32)
        for k in range(_NS):
            tot = tot + buf_v[k]
        out_v[...] = tot
        pltpu.sync_copy(out_v, out_hbm.at[cid])


@jax.jit
def _ranking_loss(packed):
    mesh = plsc.VectorSubcoreMesh(core_axis_name="c", subcore_axis_name="s")
    run = functools.partial(
        pl.kernel, mesh=mesh,
        out_type=jax.ShapeDtypeStruct((_NC, _L), jnp.float32),
        scratch_types=[
            pltpu.VMEM((48, _L), jnp.float32),
            pltpu.VMEM((_L,), jnp.float32),
            pltpu.VMEM((_NS, _L), jnp.float32),
            pltpu.VMEM_SHARED((_NS, _L), jnp.float32),
            pltpu.VMEM((_L,), jnp.float32),
        ],
    )(_loss_body)
    return run(packed)


def kernel(logits, labels):
    pad = 16 * _L - _N
    packed = jnp.concatenate([
        jnp.pad(logits, (0, pad)),
        jnp.pad(labels, (0, pad), constant_values=2.0),
        jnp.pad(labels, (0, pad), constant_values=-1.0),
    ]).reshape(48, _L)
    out = _ranking_loss(packed)
    return out[0, 0] + out[1, 0]


# speedup vs baseline: 1.0189x; 1.0180x over previous
import functools

import jax
import jax.numpy as jnp
from jax import lax
from jax.experimental import pallas as pl
from jax.experimental.pallas import tpu as pltpu
from jax.experimental.pallas import tpu_sc as plsc

_N = 200
_L = 16
_NC = 2
_NS = 16
_NW = _NC * _NS
_ROWS = 7
_CHUNKS = (_N + _L - 1) // _L
_TOL = 0.01
_NACC = 4


def _loss_body(packed_hbm, out_hbm, pk_v, acc_v, buf_v, part_sh, out_v):
    cid = lax.axis_index("c")
    sid = lax.axis_index("s")

    pltpu.sync_copy(packed_hbm, pk_v)

    jbase = lax.iota(jnp.int32, _L)
    sidvec = jnp.broadcast_to(sid, (_L,)).astype(jnp.int32)

    def row_body(r, accs):
        rrow = cid + 2 * r
        sl_i = pk_v[rrow].at[sidvec].get(mode="promise_in_bounds")
        lab_i = pk_v[32 + rrow].at[sidvec].get(mode="promise_in_bounds")

        accs = list(accs)
        for c in range(_CHUNKS):
            sl_j = pk_v[c]
            lab_j = pk_v[16 + c]
            d = sl_i - sl_j
            mval = jnp.maximum(jnp.sign(lab_i - lab_j - _TOL), 0.0)
            u = jnp.exp(-jnp.abs(d))
            s = u / (u + 2.0)
            s2 = s * s
            p = s * (2.0 + s2 * (2.0 / 3.0 + s2 * (2.0 / 5.0)))
            val = jnp.minimum(d, 0.0) - p
            accs[c % _NACC] = accs[c % _NACC] + mval * val
        return tuple(accs)

    zero = jnp.zeros((_L,), jnp.float32)
    accs = lax.fori_loop(0, _ROWS, row_body, (zero,) * _NACC)
    acc = (accs[0] + accs[1]) + (accs[2] + accs[3])

    for step in (1, 2, 4, 8):
        acc = acc + acc.at[jbase ^ step].get(mode="promise_in_bounds")

    acc_v[...] = acc
    pltpu.sync_copy(acc_v, part_sh.at[sid])
    plsc.subcore_barrier()

    @pl.when(sid == 0)
    def _():
        pltpu.sync_copy(part_sh, buf_v)
        tot = jnp.zeros((_L,), jnp.float32)
        for k in range(_NS):
            tot = tot + buf_v[k]
        out_v[...] = tot
        pltpu.sync_copy(out_v, out_hbm.at[cid])


@jax.jit
def _ranking_loss(packed):
    mesh = plsc.VectorSubcoreMesh(core_axis_name="c", subcore_axis_name="s")
    run = functools.partial(
        pl.kernel, mesh=mesh,
        out_type=jax.ShapeDtypeStruct((_NC, _L), jnp.float32),
        scratch_types=[
            pltpu.VMEM((48, _L), jnp.float32),
            pltpu.VMEM((_L,), jnp.float32),
            pltpu.VMEM((_NS, _L), jnp.float32),
            pltpu.VMEM_SHARED((_NS, _L), jnp.float32),
            pltpu.VMEM((_L,), jnp.float32),
        ],
    )(_loss_body)
    return run(packed)


def kernel(logits, labels):
    pad = 16 * _L - _N
    packed = jnp.concatenate([
        jnp.pad(logits, (0, pad)),
        jnp.pad(labels, (0, pad), constant_values=2.0),
        jnp.pad(labels, (0, pad), constant_values=-1.0),
    ]).reshape(48, _L)
    out = _ranking_loss(packed)
    return out[0, 0] + out[1, 0]
